# Initial kernel scaffold; baseline (speedup 1.0000x reference)
import jax
import jax.numpy as jnp
from jax.experimental import pallas as pl


def kernel(x, edge_index, W1, b1, W2, b2, Wfc, bfc):
    raise NotImplementedError("write your pallas kernel here")


# speedup vs baseline: 31.5466x; 31.5466x over previous
import functools

import jax
import jax.numpy as jnp
from jax import lax
from jax.experimental import pallas as pl
from jax.experimental.pallas import tpu as pltpu, tpu_sc as plsc

_N = 10000
_E = 320000
_D = 128
_H = 16

_NC = 2
_NS = 16
_NW = _NC * _NS
_IB = 128
_KCH = 79
_EPW = _KCH * _IB
_PADE = _NW * _EPW - _E
_NPAD = 10112
_ZR = _NPAD // _NS

_mesh = plsc.VectorSubcoreMesh(core_axis_name="c", subcore_axis_name="s")
_sc_params = pltpu.CompilerParams(use_tc_tiling_on_sc=False)


@functools.partial(
    pl.kernel,
    mesh=_mesh,
    compiler_params=_sc_params,
    out_type=jax.ShapeDtypeStruct((_NC, _NPAD, _H), jnp.float32),
    scratch_types=[
        pltpu.VMEM((_KCH, _IB), jnp.int32),
        pltpu.VMEM((_IB, _H), jnp.float32),
        pltpu.VMEM((_ZR, _H), jnp.float32),
        pltpu.VMEM_SHARED((_NPAD, _H), jnp.float32),
    ],
)
def _sc_deg(dst_hbm, out_hbm, idx_v, ones_v, zbuf_v, acc_sh):
    c = lax.axis_index("c")
    s = lax.axis_index("s")
    wid = s * _NC + c
    pltpu.sync_copy(dst_hbm.at[wid], idx_v)

    def _fill(i, carry):
        ones_v[i, :] = jnp.full((_H,), 1.0, jnp.float32)
        return carry

    lax.fori_loop(0, _IB, _fill, 0)

    def _zero(i, carry):
        zbuf_v[i, :] = jnp.zeros((_H,), jnp.float32)
        return carry

    lax.fori_loop(0, _ZR, _zero, 0)
    pltpu.sync_copy(zbuf_v, acc_sh.at[pl.ds(s * _ZR, _ZR)])
    plsc.subcore_barrier()

    def _scat(k, carry):
        pltpu.sync_copy(ones_v, acc_sh.at[idx_v.at[k]], add=True)
        return carry

    lax.fori_loop(0, _KCH, _scat, 0)
    plsc.subcore_barrier()
    pltpu.sync_copy(acc_sh.at[pl.ds(s * _ZR, _ZR)],
                    out_hbm.at[c, pl.ds(s * _ZR, _ZR)])


@functools.partial(
    pl.kernel,
    mesh=_mesh,
    compiler_params=_sc_params,
    out_type=jax.ShapeDtypeStruct((_NC, _NPAD, _H), jnp.float32),
    scratch_types=[
        pltpu.VMEM((_KCH, _IB), jnp.int32),
        pltpu.VMEM((_KCH, _IB), jnp.int32),
        pltpu.VMEM((_IB, _H), jnp.float32),
        pltpu.VMEM((_ZR, _H), jnp.float32),
        pltpu.VMEM_SHARED((_NPAD, _H), jnp.float32),
        pltpu.SemaphoreType.DMA,
    ],
)
def _sc_edge_sum(tab_hbm, src_hbm, dst_hbm, out_hbm,
                 src_v, dst_v, rows_v, zbuf_v, acc_sh, sem):
    c = lax.axis_index("c")
    s = lax.axis_index("s")
    wid = s * _NC + c
    pltpu.sync_copy(src_hbm.at[wid], src_v)
    pltpu.sync_copy(dst_hbm.at[wid], dst_v)

    def _zero(i, carry):
        zbuf_v[i, :] = jnp.zeros((_H,), jnp.float32)
        return carry

    lax.fori_loop(0, _ZR, _zero, 0)
    pltpu.sync_copy(zbuf_v, acc_sh.at[pl.ds(s * _ZR, _ZR)])
    plsc.subcore_barrier()

    def _body(k, carry):
        pltpu.async_copy(tab_hbm.at[src_v.at[k]], rows_v, sem).wait()
        pltpu.sync_copy(rows_v, acc_sh.at[dst_v.at[k]], add=True)
        return carry

    lax.fori_loop(0, _KCH, _body, 0)
    plsc.subcore_barrier()
    pltpu.sync_copy(acc_sh.at[pl.ds(s * _ZR, _ZR)],
                    out_hbm.at[c, pl.ds(s * _ZR, _ZR)])


def _tc1_body(x_ref, w_ref, d0_ref, d1_ref, xs_ref, dinv_ref):
    deg = d0_ref[...] + d1_ref[...] + 1.0
    dinv = lax.rsqrt(deg)
    xw = jnp.dot(x_ref[...], w_ref[...], preferred_element_type=jnp.float32)
    xs_ref[...] = xw * dinv
    dinv_ref[...] = dinv


def _tc2_body(s0_ref, s1_ref, xs_ref, dinv_ref, b_ref, w_ref, out_ref):
    dinv = dinv_ref[...]
    h = dinv * (s0_ref[...] + s1_ref[...] + xs_ref[...]) + b_ref[...]
    h = jnp.maximum(h, 0.0)
    out_ref[...] = jnp.dot(h, w_ref[...],
                           preferred_element_type=jnp.float32) * dinv


def _tc3_body(s0_ref, s1_ref, xs_ref, dinv_ref, b_ref, w_ref, bf_ref, out_ref):
    dinv = dinv_ref[...]
    h = dinv * (s0_ref[...] + s1_ref[...] + xs_ref[...]) + b_ref[...]
    h = jnp.maximum(h, 0.0)
    out_ref[...] = jnp.dot(h, w_ref[...],
                           preferred_element_type=jnp.float32) + bf_ref[...]


_tc1 = pl.pallas_call(
    _tc1_body,
    out_shape=(jax.ShapeDtypeStruct((_N, _H), jnp.float32),
               jax.ShapeDtypeStruct((_N, 1), jnp.float32)),
)

_tc2 = pl.pallas_call(
    _tc2_body,
    out_shape=jax.ShapeDtypeStruct((_N, _H), jnp.float32),
)

_tc3 = pl.pallas_call(
    _tc3_body,
    out_shape=jax.ShapeDtypeStruct((_N, 1), jnp.float32),
)


def kernel(x, edge_index, W1, b1, W2, b2, Wfc, bfc):
    ei = edge_index.astype(jnp.int32)
    src = ei[0]
    dst = ei[1]
    pad_src = jnp.zeros((_PADE,), jnp.int32)
    pad_dst = _N + (jnp.arange(_PADE, dtype=jnp.int32) % (_NPAD - _N))
    src_p = jnp.concatenate([src, pad_src]).reshape(_NW, _KCH, _IB)
    dst_p = jnp.concatenate([dst, pad_dst]).reshape(_NW, _KCH, _IB)

    dega = _sc_deg(dst_p)
    d0 = dega[0, :_N, 0:1]
    d1 = dega[1, :_N, 0:1]

    xs1, dinv = _tc1(x, W1, d0, d1)
    s1 = _sc_edge_sum(xs1, src_p, dst_p)
    xs2 = _tc2(s1[0, :_N], s1[1, :_N], xs1, dinv, b1.reshape(1, _H), W2)
    s2 = _sc_edge_sum(xs2, src_p, dst_p)
    out = _tc3(s2[0, :_N], s2[1, :_N], xs2, dinv, b2.reshape(1, _H),
               Wfc, bfc.reshape(1, 1))
    return out


# speedup vs baseline: 39.0935x; 1.2392x over previous
import functools

import jax
import jax.numpy as jnp
from jax import lax
from jax.experimental import pallas as pl
from jax.experimental.pallas import tpu as pltpu, tpu_sc as plsc

_N = 10000
_E = 320000
_D = 128
_H = 16

_NC = 2
_NS = 16
_NW = _NC * _NS
_IB = 128
_KCH = 79
_EPW = _KCH * _IB
_PADE = _NW * _EPW - _E
_NPAD = 10112
_ZR = _NPAD // _NS

_mesh = plsc.VectorSubcoreMesh(core_axis_name="c", subcore_axis_name="s")
_sc_params = pltpu.CompilerParams(use_tc_tiling_on_sc=False)


@functools.partial(
    pl.kernel,
    mesh=_mesh,
    compiler_params=_sc_params,
    out_type=jax.ShapeDtypeStruct((_NC, _NPAD, _H), jnp.float32),
    scratch_types=[
        pltpu.VMEM((_KCH, _IB), jnp.int32),
        pltpu.VMEM((_IB, _H), jnp.float32),
        pltpu.VMEM((_ZR, _H), jnp.float32),
        pltpu.VMEM_SHARED((_NPAD, _H), jnp.float32),
        pltpu.SemaphoreType.DMA,
    ],
)
def _sc_deg(dst_hbm, out_hbm, idx_v, ones_v, zbuf_v, acc_sh, sem):
    c = lax.axis_index("c")
    s = lax.axis_index("s")
    wid = s * _NC + c
    pltpu.sync_copy(dst_hbm.at[wid], idx_v)

    def _fill(i, carry):
        ones_v[i, :] = jnp.full((_H,), 1.0, jnp.float32)
        return carry

    lax.fori_loop(0, _IB, _fill, 0)

    def _zero(i, carry):
        zbuf_v[i, :] = jnp.zeros((_H,), jnp.float32)
        return carry

    lax.fori_loop(0, _ZR, _zero, 0)
    pltpu.sync_copy(zbuf_v, acc_sh.at[pl.ds(s * _ZR, _ZR)])
    plsc.subcore_barrier()

    def _scat(k, carry):
        pltpu.async_copy(ones_v, acc_sh.at[idx_v.at[k]], sem, add=True)
        return carry

    lax.fori_loop(0, _KCH, _scat, 0)

    def _drain(k, carry):
        pltpu.make_async_copy(ones_v, acc_sh.at[idx_v.at[k]], sem).wait()
        return carry

    lax.fori_loop(0, _KCH, _drain, 0)
    plsc.subcore_barrier()
    pltpu.sync_copy(acc_sh.at[pl.ds(s * _ZR, _ZR)],
                    out_hbm.at[c, pl.ds(s * _ZR, _ZR)])


@functools.partial(
    pl.kernel,
    mesh=_mesh,
    compiler_params=_sc_params,
    out_type=jax.ShapeDtypeStruct((_NC, _NPAD, _H), jnp.float32),
    scratch_types=[
        pltpu.VMEM((_KCH, _IB), jnp.int32),
        pltpu.VMEM((_KCH, _IB), jnp.int32),
        pltpu.VMEM((2, _IB, _H), jnp.float32),
        pltpu.VMEM((_ZR, _H), jnp.float32),
        pltpu.VMEM_SHARED((_NPAD, _H), jnp.float32),
        pltpu.SemaphoreType.DMA,
        pltpu.SemaphoreType.DMA,
    ],
)
def _sc_edge_sum(tab_hbm, src_hbm, dst_hbm, out_hbm,
                 src_v, dst_v, rows_v, zbuf_v, acc_sh, sem_g, sem_s):
    c = lax.axis_index("c")
    s = lax.axis_index("s")
    wid = s * _NC + c
    pltpu.sync_copy(src_hbm.at[wid], src_v)
    pltpu.sync_copy(dst_hbm.at[wid], dst_v)

    def _zero(i, carry):
        zbuf_v[i, :] = jnp.zeros((_H,), jnp.float32)
        return carry

    lax.fori_loop(0, _ZR, _zero, 0)
    pltpu.sync_copy(zbuf_v, acc_sh.at[pl.ds(s * _ZR, _ZR)])
    plsc.subcore_barrier()

    pltpu.async_copy(tab_hbm.at[src_v.at[0]], rows_v.at[0], sem_g)

    def _body(k, carry):
        b = lax.rem(k, 2)

        @pl.when(k >= 1)
        def _wait_prev_scatter():
            pltpu.make_async_copy(rows_v.at[1 - b],
                                  acc_sh.at[dst_v.at[k - 1]], sem_s).wait()

        @pl.when(k + 1 < _KCH)
        def _issue_next_gather():
            pltpu.async_copy(tab_hbm.at[src_v.at[k + 1]],
                             rows_v.at[1 - b], sem_g)

        pltpu.make_async_copy(tab_hbm.at[src_v.at[k]],
                              rows_v.at[b], sem_g).wait()
        pltpu.async_copy(rows_v.at[b], acc_sh.at[dst_v.at[k]], sem_s,
                         add=True)
        return carry

    lax.fori_loop(0, _KCH, _body, 0)
    pltpu.make_async_copy(rows_v.at[(_KCH - 1) % 2],
                          acc_sh.at[dst_v.at[_KCH - 1]], sem_s).wait()
    plsc.subcore_barrier()
    pltpu.sync_copy(acc_sh.at[pl.ds(s * _ZR, _ZR)],
                    out_hbm.at[c, pl.ds(s * _ZR, _ZR)])


def _tc1_body(x_ref, w_ref, d0_ref, d1_ref, xs_ref, dinv_ref):
    deg = d0_ref[...] + d1_ref[...] + 1.0
    dinv = lax.rsqrt(deg)
    xw = jnp.dot(x_ref[...], w_ref[...], preferred_element_type=jnp.float32)
    xs_ref[...] = xw * dinv
    dinv_ref[...] = dinv


def _tc2_body(s0_ref, s1_ref, xs_ref, dinv_ref, b_ref, w_ref, out_ref):
    dinv = dinv_ref[...]
    h = dinv * (s0_ref[...] + s1_ref[...] + xs_ref[...]) + b_ref[...]
    h = jnp.maximum(h, 0.0)
    out_ref[...] = jnp.dot(h, w_ref[...],
                           preferred_element_type=jnp.float32) * dinv


def _tc3_body(s0_ref, s1_ref, xs_ref, dinv_ref, b_ref, w_ref, bf_ref, out_ref):
    dinv = dinv_ref[...]
    h = dinv * (s0_ref[...] + s1_ref[...] + xs_ref[...]) + b_ref[...]
    h = jnp.maximum(h, 0.0)
    out_ref[...] = jnp.dot(h, w_ref[...],
                           preferred_element_type=jnp.float32) + bf_ref[...]


_tc1 = pl.pallas_call(
    _tc1_body,
    out_shape=(jax.ShapeDtypeStruct((_N, _H), jnp.float32),
               jax.ShapeDtypeStruct((_N, 1), jnp.float32)),
)

_tc2 = pl.pallas_call(
    _tc2_body,
    out_shape=jax.ShapeDtypeStruct((_N, _H), jnp.float32),
)

_tc3 = pl.pallas_call(
    _tc3_body,
    out_shape=jax.ShapeDtypeStruct((_N, 1), jnp.float32),
)


def kernel(x, edge_index, W1, b1, W2, b2, Wfc, bfc):
    ei = edge_index.astype(jnp.int32)
    src = ei[0]
    dst = ei[1]
    pad_src = jnp.zeros((_PADE,), jnp.int32)
    pad_dst = _N + (jnp.arange(_PADE, dtype=jnp.int32) % (_NPAD - _N))
    src_p = jnp.concatenate([src, pad_src]).reshape(_NW, _KCH, _IB)
    dst_p = jnp.concatenate([dst, pad_dst]).reshape(_NW, _KCH, _IB)

    dega = _sc_deg(dst_p)
    d0 = dega[0, :_N, 0:1]
    d1 = dega[1, :_N, 0:1]

    xs1, dinv = _tc1(x, W1, d0, d1)
    s1 = _sc_edge_sum(xs1, src_p, dst_p)
    xs2 = _tc2(s1[0, :_N], s1[1, :_N], xs1, dinv, b1.reshape(1, _H), W2)
    s2 = _sc_edge_sum(xs2, src_p, dst_p)
    out = _tc3(s2[0, :_N], s2[1, :_N], xs2, dinv, b2.reshape(1, _H),
               Wfc, bfc.reshape(1, 1))
    return out
